# Initial kernel scaffold; baseline (speedup 1.0000x reference)
import jax
import jax.numpy as jnp
from jax.experimental import pallas as pl


def kernel(positions, spatio_table, temporal_table):
    raise NotImplementedError("write your pallas kernel here")


# speedup vs baseline: 2.6620x; 2.6620x over previous
import functools

import jax
import jax.numpy as jnp
from jax import lax
from jax.experimental import pallas as pl
from jax.experimental.pallas import tpu as pltpu
from jax.experimental.pallas import tpu_sc as plsc

_NUM_S = 256
_NUM_T = 32
_D = 1024


def _build_body(spatio_ref, temporal_ref, out_ref):
    t = pl.program_id(0)
    row = temporal_ref[t, :]
    out_ref[...] = spatio_ref[...][None, :, :] + row[None, None, :]


def _build_combined(spatio, temporal):
    out = pl.pallas_call(
        _build_body,
        grid=(_NUM_T,),
        in_specs=[
            pl.BlockSpec((_NUM_S, _D), lambda t: (0, 0)),
            pl.BlockSpec((_NUM_T, _D), lambda t: (0, 0)),
        ],
        out_specs=pl.BlockSpec((1, _NUM_S, _D), lambda t: (t, 0, 0)),
        out_shape=jax.ShapeDtypeStruct((_NUM_T, _NUM_S, _D), jnp.float32),
    )(spatio, temporal)
    return out.reshape(_NUM_T * _NUM_S, _D)


def _gather_rows(combined, pos_chunks, n_rows):
    info = plsc.get_sparse_core_info()
    nw = info.num_cores * info.num_subcores
    bpw = n_rows // nw
    n_chunks, c = pos_chunks.shape[1], pos_chunks.shape[2]

    mesh = plsc.VectorSubcoreMesh(core_axis_name="c", subcore_axis_name="s")

    @functools.partial(
        pl.kernel,
        mesh=mesh,
        out_type=jax.ShapeDtypeStruct((n_rows, _D), jnp.float32),
        scratch_types=[
            pltpu.VMEM((n_chunks, c), jnp.int32),
            pltpu.VMEM((c, _D), jnp.float32),
            pltpu.SemaphoreType.DMA,
        ],
    )
    def k(comb_hbm, pos_hbm, out_hbm, idx_v, buf, sem):
        wid = lax.axis_index("s") * info.num_cores + lax.axis_index("c")
        base = wid * bpw
        pltpu.sync_copy(pos_hbm.at[wid], idx_v)

        def chunk(j, carry):
            pltpu.async_copy(comb_hbm.at[idx_v.at[j]], buf, sem).wait()
            pltpu.sync_copy(buf, out_hbm.at[pl.ds(base + j * c, c)])
            return carry

        lax.fori_loop(0, n_chunks, chunk, 0)

    return k(combined, pos_chunks)


def kernel(positions, spatio_table, temporal_table):
    combined = _build_combined(spatio_table, temporal_table)
    n_rows = positions.size
    c = 32
    pos_chunks = positions.reshape(32, n_rows // (32 * c), c).astype(jnp.int32)
    out = _gather_rows(combined, pos_chunks, n_rows)
    return out.reshape(positions.shape + (_D,))


# speedup vs baseline: 3.0032x; 1.1282x over previous
import functools

import jax
import jax.numpy as jnp
from jax import lax
from jax.experimental import pallas as pl
from jax.experimental.pallas import tpu as pltpu
from jax.experimental.pallas import tpu_sc as plsc

_NUM_S = 256
_NUM_T = 32
_D = 1024


def _build_body(spatio_ref, temporal_ref, out_ref):
    t = pl.program_id(0)
    row = temporal_ref[t, :]
    out_ref[...] = spatio_ref[...][None, :, :] + row[None, None, :]


def _build_combined(spatio, temporal):
    out = pl.pallas_call(
        _build_body,
        grid=(_NUM_T,),
        in_specs=[
            pl.BlockSpec((_NUM_S, _D), lambda t: (0, 0)),
            pl.BlockSpec((_NUM_T, _D), lambda t: (0, 0)),
        ],
        out_specs=pl.BlockSpec((1, _NUM_S, _D), lambda t: (t, 0, 0)),
        out_shape=jax.ShapeDtypeStruct((_NUM_T, _NUM_S, _D), jnp.float32),
    )(spatio, temporal)
    return out.reshape(_NUM_T * _NUM_S, _D)


def _gather_rows(combined, pos_chunks, n_rows):
    info = plsc.get_sparse_core_info()
    nw = info.num_cores * info.num_subcores
    bpw = n_rows // nw
    n_chunks, c = pos_chunks.shape[1], pos_chunks.shape[2]

    mesh = plsc.VectorSubcoreMesh(core_axis_name="c", subcore_axis_name="s")

    @functools.partial(
        pl.kernel,
        mesh=mesh,
        out_type=jax.ShapeDtypeStruct((n_rows, _D), jnp.float32),
        scratch_types=[
            pltpu.VMEM((n_chunks, c), jnp.int32),
            pltpu.VMEM((c, _D), jnp.float32),
            pltpu.VMEM((c, _D), jnp.float32),
            pltpu.SemaphoreType.DMA,
            pltpu.SemaphoreType.DMA,
            pltpu.SemaphoreType.DMA,
            pltpu.SemaphoreType.DMA,
        ],
    )
    def k(comb_hbm, pos_hbm, out_hbm, idx_v, buf0, buf1, gs0, gs1, os0, os1):
        wid = lax.axis_index("s") * info.num_cores + lax.axis_index("c")
        base = wid * bpw
        pltpu.sync_copy(pos_hbm.at[wid], idx_v)

        bufs, gs, osm = (buf0, buf1), (gs0, gs1), (os0, os1)

        def gather_desc(j, b):
            return pltpu.make_async_copy(comb_hbm.at[idx_v.at[j]], bufs[b], gs[b])

        def out_desc(j, b):
            return pltpu.make_async_copy(
                bufs[b], out_hbm.at[pl.ds(base + j * c, c)], osm[b]
            )

        gather_desc(0, 0).start()

        def g_body(g, carry):
            for b in (0, 1):
                j = 2 * g + b
                gather_desc(j, b).wait()
                out_desc(j, b).start()

                @pl.when(j < n_chunks - 1)
                def _():
                    @pl.when(j >= 1)
                    def _():
                        out_desc(j - 1, 1 - b).wait()

                    gather_desc(j + 1, 1 - b).start()

            return carry

        lax.fori_loop(0, n_chunks // 2, g_body, 0)
        out_desc(n_chunks - 2, 0).wait()
        out_desc(n_chunks - 1, 1).wait()

    return k(combined, pos_chunks)


def kernel(positions, spatio_table, temporal_table):
    combined = _build_combined(spatio_table, temporal_table)
    n_rows = positions.size
    c = 32
    pos_chunks = positions.reshape(32, n_rows // (32 * c), c).astype(jnp.int32)
    out = _gather_rows(combined, pos_chunks, n_rows)
    return out.reshape(positions.shape + (_D,))
